# Initial kernel scaffold; baseline (speedup 1.0000x reference)
import jax
import jax.numpy as jnp
from jax.experimental import pallas as pl


def kernel(X1, L1_idx, L1_val, Lu_idx, Ld_idx, batch1, Wp1, Wu1, asu1, adu1, Wd1, asd1, add1, Wp2, Wu2, asu2, adu2, Wd2, asd2, add2, Wp3, Wu3, asu3, adu3, Wd3, asd3, add3, Wp4, Wu4, asu4, adu4, Wd4, asd4, add4):
    raise NotImplementedError("write your pallas kernel here")


# speedup vs baseline: 15.2735x; 15.2735x over previous
import functools

import jax
import jax.numpy as jnp
from jax import lax
from jax.experimental import pallas as pl
from jax.experimental.pallas import tpu as pltpu
from jax.experimental.pallas import tpu_sc as plsc

N = 10000
E = 320000
D_IN = 128
F = 32
F4 = 16
OUT = 10
B = 16

NC = 2
NS = 16
LANES = 16

NP = 10240
NSL = NP // NS
EPW = E // (NC * NS)
C = 80
NCH = EPW // C
NEG = -1e30

_MESH = plsc.VectorSubcoreMesh(
    core_axis_name="c", subcore_axis_name="s", num_cores=NC, num_subcores=NS)
_SC_PARAMS = pltpu.CompilerParams(
    needs_layout_passes=False, use_tc_tiling_on_sc=False)


def _f32(shape):
    return jax.ShapeDtypeStruct(shape, jnp.float32)


def _seg_max_scatter(mloc, d16, e):
    lanes = lax.iota(jnp.int32, LANES)
    k, v = plsc.sort_key_val(d16, e)
    for step in (1, 2, 4, 8):
        idx = jnp.maximum(lanes - step, 0)
        kk = k.at[idx].get(mode="promise_in_bounds")
        vv = v.at[idx].get(mode="promise_in_bounds")
        v = jnp.where(kk == k, jnp.maximum(v, vv), v)
    nxt = k.at[jnp.minimum(lanes + 1, LANES - 1)].get(
        mode="promise_in_bounds")
    isend = (k != nxt) | (lanes == LANES - 1)
    old = plsc.load_gather(mloc, [k], mask=isend)
    plsc.store_scatter(mloc, [k], jnp.maximum(old, v), mask=isend)


def _make_layer(f):
    @functools.partial(
        pl.kernel,
        out_type=(
            _f32((NC, NP, f)),
            _f32((NC, NS, NP)),
            _f32((NC, NP, f)),
            _f32((NC, NS, NP)),
            _f32((NC, NP, f)),
        ),
        mesh=_MESH,
        compiler_params=_SC_PARAMS,
        scratch_types=dict(
            sa_v=pltpu.VMEM((NP,), jnp.float32),
            sb_v=pltpu.VMEM((NP,), jnp.float32),
            mloc=pltpu.VMEM((NP,), jnp.float32),
            sloc=pltpu.VMEM((NP,), jnp.float32),
            esrc=pltpu.VMEM((EPW,), jnp.int32),
            edst=pltpu.VMEM((EPW,), jnp.int32),
            vbuf=pltpu.VMEM((EPW,), jnp.float32),
            gbuf=pltpu.VMEM((2, C, f), jnp.float32),
            sbuf=pltpu.VMEM((2, C, f), jnp.float32),
            wbuf=pltpu.VMEM((C,), jnp.float32),
            se0=pltpu.VMEM((C,), jnp.int32),
            se1=pltpu.VMEM((C,), jnp.int32),
            sg0=pltpu.VMEM((C,), jnp.int32),
            sg1=pltpu.VMEM((C,), jnp.int32),
            red=pltpu.VMEM((NS, NSL), jnp.float32),
            stg=pltpu.VMEM((NSL,), jnp.float32),
            acc_s=pltpu.VMEM_SHARED((NP, f), jnp.float32),
            sh=pltpu.VMEM_SHARED((NS, NP), jnp.float32),
            gsem0=pltpu.SemaphoreType.DMA,
            gsem1=pltpu.SemaphoreType.DMA,
            ssem0=pltpu.SemaphoreType.DMA,
            ssem1=pltpu.SemaphoreType.DMA,
        ),
    )
    def layer(su_h, du_h, sd_h, dd_h, hu_h, hd_h, hp_h,
              usrc_h, udst_h, dsrc_h, ddst_h, psrc_h, pdst_h, pval_h,
              accu_h, ssu_h, accd_h, ssd_h, accp_h, *,
              sa_v, sb_v, mloc, sloc, esrc, edst, vbuf, gbuf, sbuf, wbuf,
              se0, se1, sg0, sg1, red, stg, acc_s, sh,
              gsem0, gsem1, ssem0, ssem1):
        c = lax.axis_index("c")
        s = lax.axis_index("s")
        ebase = (c * NS + s) * EPW
        gsems = (gsem0, gsem1)
        ssems = (ssem0, ssem1)
        sg = (sg0, sg1)
        se = (se0, se1)

        def zero_np(ref):
            @pl.loop(0, NP // LANES)
            def _z(i):
                ref[pl.ds(i * LANES, LANES)] = jnp.zeros((LANES,), jnp.float32)

        def edge_vec(j):
            s16 = esrc[pl.ds(j * LANES, LANES)]
            d16 = edst[pl.ds(j * LANES, LANES)]
            return s16, d16

        def logits(s16, d16):
            t = (plsc.load_gather(sa_v, [s16])
                 + plsc.load_gather(sb_v, [d16]))
            return jnp.maximum(t, 0.2 * t)

        def max_phase(src_h, dst_h):
            @pl.loop(0, NP // LANES)
            def _zm(i):
                mloc[pl.ds(i * LANES, LANES)] = jnp.full(
                    (LANES,), NEG, jnp.float32)

            for cc in range(NC):
                base = (cc * NS) * EPW
                pltpu.sync_copy(
                    src_h.at[pl.ds(base + s * EPW, EPW)], esrc)
                pltpu.sync_copy(
                    dst_h.at[pl.ds(base + s * EPW, EPW)], edst)

                @pl.loop(0, EPW // LANES)
                def _scan(j):
                    s16, d16 = edge_vec(j)
                    _seg_max_scatter(mloc, d16, logits(s16, d16))

            pltpu.sync_copy(mloc, sh.at[s])
            plsc.subcore_barrier()
            for k in range(NS):
                pltpu.sync_copy(sh.at[k, pl.ds(s * NSL, NSL)], red.at[k])

            @pl.loop(0, NSL // LANES)
            def _fin(i):
                acc = red[0, pl.ds(i * LANES, LANES)]
                for k in range(1, NS):
                    acc = jnp.maximum(acc, red[k, pl.ds(i * LANES, LANES)])
                stg[pl.ds(i * LANES, LANES)] = acc

            plsc.subcore_barrier()
            pltpu.sync_copy(stg, sh.at[0, pl.ds(s * NSL, NSL)])
            plsc.subcore_barrier()
            pltpu.sync_copy(sh.at[0], mloc)

        def zero_acc():
            @pl.loop(0, C)
            def _zb(i):
                for half in range(f // LANES):
                    sbuf[0, i, pl.ds(half * LANES, LANES)] = jnp.zeros(
                        (LANES,), jnp.float32)

            for k in range(NSL // C):
                pltpu.sync_copy(sbuf.at[0],
                                acc_s.at[pl.ds(s * NSL + k * C, C)])
            plsc.subcore_barrier()

        def read_acc(out_h):
            plsc.subcore_barrier()
            for k in range(NSL // C):
                sl = pl.ds(s * NSL + k * C, C)
                pltpu.sync_copy(acc_s.at[sl], gbuf.at[0])
                pltpu.sync_copy(gbuf.at[0], out_h.at[c, sl, :])
            plsc.subcore_barrier()

        def row_pass(src_h, dst_h, h_h, is_gat):
            def issue_gather(ci, b):
                pltpu.async_copy(
                    h_h.at[esrc.at[pl.ds(ci * C, C)]], gbuf.at[b], gsems[b])

            def wait_gather(b):
                pltpu.make_async_copy(
                    h_h.at[esrc.at[pl.ds(0, C)]], gbuf.at[b],
                    gsems[b]).wait()

            def stage_idx(ci, b):
                off = ebase + ci * C
                pltpu.sync_copy(src_h.at[pl.ds(off, C)], sg[b])
                pltpu.sync_copy(dst_h.at[pl.ds(off, C)], se[b])

            def compute(ci, b):
                for j in range(C // LANES):
                    sl16 = pl.ds(ci * C + j * LANES, LANES)
                    s16 = esrc[sl16]
                    d16 = edst[sl16]
                    if is_gat:
                        e = logits(s16, d16)
                        w = jnp.exp(e - plsc.load_gather(mloc, [d16]))
                        plsc.addupdate_scatter(sloc, [d16], w)
                    else:
                        w = vbuf[sl16]
                    wbuf[pl.ds(j * LANES, LANES)] = w

                for j in range(C // LANES):
                    w16 = wbuf[pl.ds(j * LANES, LANES)]
                    for r in range(LANES):
                        jj = j * LANES + r
                        wj = w16[r]
                        for half in range(f // LANES):
                            fl = pl.ds(half * LANES, LANES)
                            sbuf[b, jj, fl] = gbuf[b, jj, fl] * wj

            def issue_scatter(b):
                pltpu.sync_copy(sbuf.at[b], acc_s.at[se[b]], add=True)

            @pl.loop(0, NCH)
            def _chunks(ci):
                stage_idx(ci, 0)
                pltpu.async_copy(
                    h_h.at[sg[0]], gbuf.at[0], gsems[0]).wait()
                compute(ci, 0)
                issue_scatter(0)

        def gat_set(src_h, dst_h, su2_h, du2_h, h_h, acc_h, sp_h):
            pltpu.sync_copy(su2_h, sa_v)
            pltpu.sync_copy(du2_h, sb_v)
            max_phase(src_h, dst_h)
            pltpu.sync_copy(src_h.at[pl.ds(ebase, EPW)], esrc)
            pltpu.sync_copy(dst_h.at[pl.ds(ebase, EPW)], edst)
            zero_np(sloc)
            zero_acc()
            row_pass(src_h, dst_h, h_h, True)
            pltpu.sync_copy(sloc, sp_h.at[c, s])
            read_acc(acc_h)

        gat_set(usrc_h, udst_h, su_h, du_h, hu_h, accu_h, ssu_h)
        gat_set(dsrc_h, ddst_h, sd_h, dd_h, hd_h, accd_h, ssd_h)

        pltpu.sync_copy(psrc_h.at[pl.ds(ebase, EPW)], esrc)
        pltpu.sync_copy(pdst_h.at[pl.ds(ebase, EPW)], edst)
        pltpu.sync_copy(pval_h.at[pl.ds(ebase, EPW)], vbuf)
        zero_acc()
        row_pass(psrc_h, pdst_h, hp_h, False)
        read_acc(accp_h)

    return layer


_BLK = 2048


def _prep_compute(x, Wp, Wu, asu, adu, Wd, asd, addv):
    hp = x @ Wp
    hu = x @ Wu
    hd = x @ Wd
    return (hp, hu, hd, hu @ asu, hu @ adu, hd @ asd, hd @ addv)


_W2 = lambda shape: pl.BlockSpec(shape, lambda i: (0, 0))
_W1 = lambda shape: pl.BlockSpec(shape, lambda i: (0,))


def _prep_out_specs(do):
    return (
        [pl.BlockSpec((_BLK, do), lambda i: (i, 0))] * 3
        + [pl.BlockSpec((_BLK,), lambda i: (i,))] * 4,
        [_f32((NP, do))] * 3 + [_f32((NP,))] * 4,
    )


def _make_prep(di, do):
    def body(x_ref, wp_ref, wu_ref, asu_ref, adu_ref, wd_ref, asd_ref,
             add_ref, hp_ref, hu_ref, hd_ref, su_ref, du_ref, sd_ref, dd_ref):
        outs = _prep_compute(x_ref[...], wp_ref[...], wu_ref[...],
                             asu_ref[...], adu_ref[...], wd_ref[...],
                             asd_ref[...], add_ref[...])
        for ref, v in zip(
                (hp_ref, hu_ref, hd_ref, su_ref, du_ref, sd_ref, dd_ref),
                outs):
            ref[...] = v

    out_specs, out_shape = _prep_out_specs(do)
    return pl.pallas_call(
        body,
        grid=(NP // _BLK,),
        in_specs=[
            pl.BlockSpec((_BLK, di), lambda i: (i, 0)),
            _W2((di, do)), _W2((di, do)), _W1((do,)), _W1((do,)),
            _W2((di, do)), _W1((do,)), _W1((do,)),
        ],
        out_specs=out_specs,
        out_shape=out_shape,
    )


def _combine_x(au_ref, su_ref, ad_ref, sd_ref, ap_ref):
    ssu = jnp.sum(su_ref[...], axis=(0, 1))
    ssd = jnp.sum(sd_ref[...], axis=(0, 1))
    hu = (au_ref[0] + au_ref[1]) / (ssu + 1e-16)[:, None]
    hd = (ad_ref[0] + ad_ref[1]) / (ssd + 1e-16)[:, None]
    hp = ap_ref[0] + ap_ref[1]
    return jax.nn.relu(hu + hd + hp)


def _make_combine_prep(fin, do):
    def body(au_ref, su_ref, ad_ref, sd_ref, ap_ref,
             wp_ref, wu_ref, asu_ref, adu_ref, wd_ref, asd_ref, add_ref,
             hp_ref, hu_ref, hd_ref, su_o, du_o, sd_o, dd_o):
        x = _combine_x(au_ref, su_ref, ad_ref, sd_ref, ap_ref)
        outs = _prep_compute(x, wp_ref[...], wu_ref[...], asu_ref[...],
                             adu_ref[...], wd_ref[...], asd_ref[...],
                             add_ref[...])
        for ref, v in zip((hp_ref, hu_ref, hd_ref, su_o, du_o, sd_o, dd_o),
                          outs):
            ref[...] = v

    aspec = pl.BlockSpec((NC, _BLK, fin), lambda i: (0, i, 0))
    sspec = pl.BlockSpec((NC, NS, _BLK), lambda i: (0, 0, i))
    out_specs, out_shape = _prep_out_specs(do)
    return pl.pallas_call(
        body,
        grid=(NP // _BLK,),
        in_specs=[
            aspec, sspec, aspec, sspec, aspec,
            _W2((fin, do)), _W2((fin, do)), _W1((do,)), _W1((do,)),
            _W2((fin, do)), _W1((do,)), _W1((do,)),
        ],
        out_specs=out_specs,
        out_shape=out_shape,
    )


def _make_pool():
    def body(au_ref, su_ref, ad_ref, sd_ref, ap_ref, b_ref, out_ref):
        xa = jnp.abs(_combine_x(au_ref, su_ref, ad_ref, sd_ref, ap_ref))
        bid = b_ref[...]
        rows = lax.broadcasted_iota(jnp.int32, (B, NP), 0)
        onehot = (bid[None, :] == rows).astype(jnp.float32)
        sums = onehot @ xa
        cnt = jnp.sum(onehot, axis=1)
        pooled = sums / jnp.maximum(cnt, 1.0)[:, None]
        cols = lax.broadcasted_iota(jnp.int32, (B, F4), 1)
        logits = jnp.where(cols < OUT, pooled, -1e30)
        mx = jnp.max(logits, axis=1, keepdims=True)
        ex = jnp.exp(logits - mx)
        out_ref[...] = ex / jnp.sum(ex, axis=1, keepdims=True)

    return pl.pallas_call(
        body,
        in_specs=[
            pl.BlockSpec((NC, NP, F4), lambda: (0, 0, 0)),
            pl.BlockSpec((NC, NS, NP), lambda: (0, 0, 0)),
            pl.BlockSpec((NC, NP, F4), lambda: (0, 0, 0)),
            pl.BlockSpec((NC, NS, NP), lambda: (0, 0, 0)),
            pl.BlockSpec((NC, NP, F4), lambda: (0, 0, 0)),
            pl.BlockSpec((NP,), lambda: (0,)),
        ],
        out_specs=pl.BlockSpec((B, F4), lambda: (0, 0)),
        out_shape=_f32((B, F4)),
    )


_layer32 = _make_layer(F)
_layer16 = _make_layer(F4)
_prep1 = _make_prep(D_IN, F)
_combine23 = _make_combine_prep(F, F)
_combine4 = _make_combine_prep(F, F4)
_pool = _make_pool()


def kernel(X1, L1_idx, L1_val, Lu_idx, Ld_idx, batch1,
           Wp1, Wu1, asu1, adu1, Wd1, asd1, add1,
           Wp2, Wu2, asu2, adu2, Wd2, asd2, add2,
           Wp3, Wu3, asu3, adu3, Wd3, asd3, add3,
           Wp4, Wu4, asu4, adu4, Wd4, asd4, add4):
    pad_n = NP - N
    Xp = jnp.pad(X1, ((0, pad_n), (0, 0)))
    bp = jnp.pad(batch1, (0, pad_n), constant_values=-1)

    usrc, udst = Lu_idx[0], Lu_idx[1]
    dsrc, ddst = Ld_idx[0], Ld_idx[1]
    psrc, pdst = L1_idx[1], L1_idx[0]

    pw = lambda w: jnp.pad(w, ((0, 0), (0, F4 - OUT)))
    pa = lambda a: jnp.pad(a, (0, F4 - OUT))
    w4 = (pw(Wp4), pw(Wu4), pa(asu4), pa(adu4), pw(Wd4), pa(asd4), pa(add4))

    hp, hu, hd, su, du, sd, dd = _prep1(
        Xp, Wp1, Wu1, asu1, adu1, Wd1, asd1, add1)

    weights = [
        (Wp2, Wu2, asu2, adu2, Wd2, asd2, add2),
        (Wp3, Wu3, asu3, adu3, Wd3, asd3, add3),
        w4,
    ]

    for l in range(4):
        lay = _layer16 if l == 3 else _layer32
        accu, ssu, accd, ssd, accp = lay(
            su, du, sd, dd, hu, hd, hp,
            usrc, udst, dsrc, ddst, psrc, pdst, L1_val)
        if l < 2:
            hp, hu, hd, su, du, sd, dd = _combine23(
                accu, ssu, accd, ssd, accp, *weights[l])
        elif l == 2:
            hp, hu, hd, su, du, sd, dd = _combine4(
                accu, ssu, accd, ssd, accp, *weights[l])
        else:
            out16 = _pool(accu, ssu, accd, ssd, accp, bp)

    return out16[:, :OUT]


# speedup vs baseline: 35.6352x; 2.3331x over previous
import functools

import jax
import jax.numpy as jnp
from jax import lax
from jax.experimental import pallas as pl
from jax.experimental.pallas import tpu as pltpu
from jax.experimental.pallas import tpu_sc as plsc

N = 10000
E = 320000
D_IN = 128
F = 32
F4 = 16
OUT = 10
B = 16

NC = 2
NS = 16
LANES = 16

NP = 10240
NSL = NP // NS
EPW = E // (NC * NS)
C = 80
NCH = EPW // C
NEG = -1e30

_MESH = plsc.VectorSubcoreMesh(
    core_axis_name="c", subcore_axis_name="s", num_cores=NC, num_subcores=NS)
_SC_PARAMS = pltpu.CompilerParams(
    needs_layout_passes=False, use_tc_tiling_on_sc=False)


def _f32(shape):
    return jax.ShapeDtypeStruct(shape, jnp.float32)


def _seg_max_scatter(mloc, d16, e):
    lanes = lax.iota(jnp.int32, LANES)
    k, v = plsc.sort_key_val(d16, e)
    for step in (1, 2, 4, 8):
        idx = jnp.maximum(lanes - step, 0)
        kk = k.at[idx].get(mode="promise_in_bounds")
        vv = v.at[idx].get(mode="promise_in_bounds")
        v = jnp.where(kk == k, jnp.maximum(v, vv), v)
    nxt = k.at[jnp.minimum(lanes + 1, LANES - 1)].get(
        mode="promise_in_bounds")
    isend = (k != nxt) | (lanes == LANES - 1)
    old = plsc.load_gather(mloc, [k], mask=isend)
    plsc.store_scatter(mloc, [k], jnp.maximum(old, v), mask=isend)


def _make_layer(f):
    @functools.partial(
        pl.kernel,
        out_type=(
            _f32((NC, NP, f)),
            _f32((NC, NS, NP)),
            _f32((NC, NP, f)),
            _f32((NC, NS, NP)),
            _f32((NC, NP, f)),
        ),
        mesh=_MESH,
        compiler_params=_SC_PARAMS,
        scratch_types=dict(
            sa_v=pltpu.VMEM((NP,), jnp.float32),
            sb_v=pltpu.VMEM((NP,), jnp.float32),
            mloc=pltpu.VMEM((NP,), jnp.float32),
            sloc=pltpu.VMEM((NP,), jnp.float32),
            esrc=pltpu.VMEM((NCH, C), jnp.int32),
            edst=pltpu.VMEM((NCH, C), jnp.int32),
            vbuf=pltpu.VMEM((NCH, C), jnp.float32),
            gbuf=pltpu.VMEM((2, C, f), jnp.float32),
            sbuf=pltpu.VMEM((2, C, f), jnp.float32),
            wbuf=pltpu.VMEM((C,), jnp.float32),
            red=pltpu.VMEM((NS, NSL), jnp.float32),
            stg=pltpu.VMEM((NSL,), jnp.float32),
            acc_s=pltpu.VMEM_SHARED((NP, f), jnp.float32),
            sh=pltpu.VMEM_SHARED((NS, NP), jnp.float32),
            gsem0=pltpu.SemaphoreType.DMA,
            gsem1=pltpu.SemaphoreType.DMA,
            ssem0=pltpu.SemaphoreType.DMA,
            ssem1=pltpu.SemaphoreType.DMA,
        ),
    )
    def layer(su_h, du_h, sd_h, dd_h, hu_h, hd_h, hp_h,
              usrc_h, udst_h, dsrc_h, ddst_h, psrc_h, pdst_h, pval_h,
              accu_h, ssu_h, accd_h, ssd_h, accp_h, *,
              sa_v, sb_v, mloc, sloc, esrc, edst, vbuf, gbuf, sbuf, wbuf,
              red, stg, acc_s, sh, gsem0, gsem1, ssem0, ssem1):
        c = lax.axis_index("c")
        s = lax.axis_index("s")
        ebase = (c * NS + s) * EPW
        gsems = (gsem0, gsem1)
        ssems = (ssem0, ssem1)

        def zero_np(ref):
            @pl.loop(0, NP // LANES)
            def _z(i):
                ref[pl.ds(i * LANES, LANES)] = jnp.zeros((LANES,), jnp.float32)

        def load_eset(src_h, dst_h, w):
            pltpu.sync_copy(src_h.at[pl.ds(w * NCH, NCH), :], esrc)
            pltpu.sync_copy(dst_h.at[pl.ds(w * NCH, NCH), :], edst)

        def logits(s16, d16):
            t = (plsc.load_gather(sa_v, [s16])
                 + plsc.load_gather(sb_v, [d16]))
            return jnp.maximum(t, 0.2 * t)

        def max_phase(src_h, dst_h):
            @pl.loop(0, NP // LANES)
            def _zm(i):
                mloc[pl.ds(i * LANES, LANES)] = jnp.full(
                    (LANES,), NEG, jnp.float32)

            for cc in range(NC):
                load_eset(src_h, dst_h, cc * NS + s)

                @pl.loop(0, NCH)
                def _scanc(ci):
                    for j in range(C // LANES):
                        sl16 = pl.ds(j * LANES, LANES)
                        s16 = esrc[ci, sl16]
                        d16 = edst[ci, sl16]
                        _seg_max_scatter(mloc, d16, logits(s16, d16))

            pltpu.sync_copy(mloc, sh.at[s])
            plsc.subcore_barrier()
            for k in range(NS):
                pltpu.sync_copy(sh.at[k, pl.ds(s * NSL, NSL)], red.at[k])

            @pl.loop(0, NSL // LANES)
            def _fin(i):
                acc = red[0, pl.ds(i * LANES, LANES)]
                for k in range(1, NS):
                    acc = jnp.maximum(acc, red[k, pl.ds(i * LANES, LANES)])
                stg[pl.ds(i * LANES, LANES)] = acc

            plsc.subcore_barrier()
            pltpu.sync_copy(stg, sh.at[0, pl.ds(s * NSL, NSL)])
            plsc.subcore_barrier()
            pltpu.sync_copy(sh.at[0], mloc)

        def zero_acc():
            @pl.loop(0, C)
            def _zb(i):
                for half in range(f // LANES):
                    sbuf[0, i, pl.ds(half * LANES, LANES)] = jnp.zeros(
                        (LANES,), jnp.float32)

            for k in range(NSL // C):
                pltpu.sync_copy(sbuf.at[0],
                                acc_s.at[pl.ds(s * NSL + k * C, C)])
            plsc.subcore_barrier()

        def read_acc(out_h):
            plsc.subcore_barrier()
            for k in range(NSL // C):
                sl = pl.ds(s * NSL + k * C, C)
                pltpu.sync_copy(acc_s.at[sl], gbuf.at[0])
                pltpu.sync_copy(gbuf.at[0], out_h.at[c, sl, :])
            plsc.subcore_barrier()

        def row_pass(h_h, is_gat):
            def issue_gather(ci, b):
                pltpu.async_copy(
                    h_h.at[esrc.at[ci]], gbuf.at[b], gsems[b])

            def wait_gather(b):
                pltpu.make_async_copy(
                    h_h.at[esrc.at[0]], gbuf.at[b], gsems[b]).wait()

            def compute(ci, b):
                for j in range(C // LANES):
                    sl16 = pl.ds(j * LANES, LANES)
                    if is_gat:
                        s16 = esrc[ci, sl16]
                        d16 = edst[ci, sl16]
                        e = logits(s16, d16)
                        w = jnp.exp(e - plsc.load_gather(mloc, [d16]))
                        plsc.addupdate_scatter(sloc, [d16], w)
                        wbuf[sl16] = w
                for j in range(C // LANES):
                    if is_gat:
                        w16 = wbuf[pl.ds(j * LANES, LANES)]
                    else:
                        w16 = vbuf[ci, pl.ds(j * LANES, LANES)]
                    for r in range(LANES):
                        jj = j * LANES + r
                        wj = w16[r]
                        for half in range(f // LANES):
                            fl = pl.ds(half * LANES, LANES)
                            sbuf[b, jj, fl] = gbuf[b, jj, fl] * wj

            def scatter(ci, b):
                pltpu.sync_copy(sbuf.at[b], acc_s.at[edst.at[ci]], add=True)

            issue_gather(0, 0)
            issue_gather(1, 1)

            @pl.loop(0, NCH - 1, step=2)
            def _chunks(i):
                for b in range(2):
                    ci = i + b
                    wait_gather(b)
                    compute(ci, b)
                    nxt = jnp.minimum(ci + 2, NCH - 1)
                    issue_gather(nxt, b)
                    scatter(ci, b)

            wait_gather(0)
            compute(NCH - 1, 0)
            scatter(NCH - 1, 0)
            wait_gather(1)

        def gat_set(src_h, dst_h, su2_h, du2_h, h_h, acc_h, sp_h):
            pltpu.sync_copy(su2_h, sa_v)
            pltpu.sync_copy(du2_h, sb_v)
            max_phase(src_h, dst_h)
            load_eset(src_h, dst_h, c * NS + s)
            zero_np(sloc)
            zero_acc()
            row_pass(h_h, True)
            pltpu.sync_copy(sloc, sp_h.at[c, s])
            read_acc(acc_h)

        gat_set(usrc_h, udst_h, su_h, du_h, hu_h, accu_h, ssu_h)
        gat_set(dsrc_h, ddst_h, sd_h, dd_h, hd_h, accd_h, ssd_h)

        wid = c * NS + s
        load_eset(psrc_h, pdst_h, wid)
        pltpu.sync_copy(pval_h.at[pl.ds(wid * NCH, NCH), :], vbuf)
        zero_acc()
        row_pass(hp_h, False)
        read_acc(accp_h)

    return layer


_BLK = 2048


def _prep_compute(x, Wp, Wu, asu, adu, Wd, asd, addv):
    hp = x @ Wp
    hu = x @ Wu
    hd = x @ Wd
    return (hp, hu, hd, hu @ asu, hu @ adu, hd @ asd, hd @ addv)


_W2 = lambda shape: pl.BlockSpec(shape, lambda i: (0, 0))
_W1 = lambda shape: pl.BlockSpec(shape, lambda i: (0,))


def _prep_out_specs(do):
    return (
        [pl.BlockSpec((_BLK, do), lambda i: (i, 0))] * 3
        + [pl.BlockSpec((_BLK,), lambda i: (i,))] * 4,
        [_f32((NP, do))] * 3 + [_f32((NP,))] * 4,
    )


def _make_prep(di, do):
    def body(x_ref, wp_ref, wu_ref, asu_ref, adu_ref, wd_ref, asd_ref,
             add_ref, hp_ref, hu_ref, hd_ref, su_ref, du_ref, sd_ref, dd_ref):
        outs = _prep_compute(x_ref[...], wp_ref[...], wu_ref[...],
                             asu_ref[...], adu_ref[...], wd_ref[...],
                             asd_ref[...], add_ref[...])
        for ref, v in zip(
                (hp_ref, hu_ref, hd_ref, su_ref, du_ref, sd_ref, dd_ref),
                outs):
            ref[...] = v

    out_specs, out_shape = _prep_out_specs(do)
    return pl.pallas_call(
        body,
        grid=(NP // _BLK,),
        in_specs=[
            pl.BlockSpec((_BLK, di), lambda i: (i, 0)),
            _W2((di, do)), _W2((di, do)), _W1((do,)), _W1((do,)),
            _W2((di, do)), _W1((do,)), _W1((do,)),
        ],
        out_specs=out_specs,
        out_shape=out_shape,
    )


def _combine_x(au_ref, su_ref, ad_ref, sd_ref, ap_ref):
    ssu = jnp.sum(su_ref[...], axis=(0, 1))
    ssd = jnp.sum(sd_ref[...], axis=(0, 1))
    hu = (au_ref[0] + au_ref[1]) / (ssu + 1e-16)[:, None]
    hd = (ad_ref[0] + ad_ref[1]) / (ssd + 1e-16)[:, None]
    hp = ap_ref[0] + ap_ref[1]
    return jax.nn.relu(hu + hd + hp)


def _make_combine_prep(fin, do):
    def body(au_ref, su_ref, ad_ref, sd_ref, ap_ref,
             wp_ref, wu_ref, asu_ref, adu_ref, wd_ref, asd_ref, add_ref,
             hp_ref, hu_ref, hd_ref, su_o, du_o, sd_o, dd_o):
        x = _combine_x(au_ref, su_ref, ad_ref, sd_ref, ap_ref)
        outs = _prep_compute(x, wp_ref[...], wu_ref[...], asu_ref[...],
                             adu_ref[...], wd_ref[...], asd_ref[...],
                             add_ref[...])
        for ref, v in zip((hp_ref, hu_ref, hd_ref, su_o, du_o, sd_o, dd_o),
                          outs):
            ref[...] = v

    aspec = pl.BlockSpec((NC, _BLK, fin), lambda i: (0, i, 0))
    sspec = pl.BlockSpec((NC, NS, _BLK), lambda i: (0, 0, i))
    out_specs, out_shape = _prep_out_specs(do)
    return pl.pallas_call(
        body,
        grid=(NP // _BLK,),
        in_specs=[
            aspec, sspec, aspec, sspec, aspec,
            _W2((fin, do)), _W2((fin, do)), _W1((do,)), _W1((do,)),
            _W2((fin, do)), _W1((do,)), _W1((do,)),
        ],
        out_specs=out_specs,
        out_shape=out_shape,
    )


def _make_pool():
    def body(au_ref, su_ref, ad_ref, sd_ref, ap_ref, b_ref, out_ref):
        xa = jnp.abs(_combine_x(au_ref, su_ref, ad_ref, sd_ref, ap_ref))
        bid = b_ref[...]
        rows = lax.broadcasted_iota(jnp.int32, (B, NP), 0)
        onehot = (bid[None, :] == rows).astype(jnp.float32)
        sums = onehot @ xa
        cnt = jnp.sum(onehot, axis=1)
        pooled = sums / jnp.maximum(cnt, 1.0)[:, None]
        cols = lax.broadcasted_iota(jnp.int32, (B, F4), 1)
        logits = jnp.where(cols < OUT, pooled, -1e30)
        mx = jnp.max(logits, axis=1, keepdims=True)
        ex = jnp.exp(logits - mx)
        out_ref[...] = ex / jnp.sum(ex, axis=1, keepdims=True)

    return pl.pallas_call(
        body,
        in_specs=[
            pl.BlockSpec((NC, NP, F4), lambda: (0, 0, 0)),
            pl.BlockSpec((NC, NS, NP), lambda: (0, 0, 0)),
            pl.BlockSpec((NC, NP, F4), lambda: (0, 0, 0)),
            pl.BlockSpec((NC, NS, NP), lambda: (0, 0, 0)),
            pl.BlockSpec((NC, NP, F4), lambda: (0, 0, 0)),
            pl.BlockSpec((NP,), lambda: (0,)),
        ],
        out_specs=pl.BlockSpec((B, F4), lambda: (0, 0)),
        out_shape=_f32((B, F4)),
    )


_layer32 = _make_layer(F)
_layer16 = _make_layer(F4)
_prep1 = _make_prep(D_IN, F)
_combine23 = _make_combine_prep(F, F)
_combine4 = _make_combine_prep(F, F4)
_pool = _make_pool()


def kernel(X1, L1_idx, L1_val, Lu_idx, Ld_idx, batch1,
           Wp1, Wu1, asu1, adu1, Wd1, asd1, add1,
           Wp2, Wu2, asu2, adu2, Wd2, asd2, add2,
           Wp3, Wu3, asu3, adu3, Wd3, asd3, add3,
           Wp4, Wu4, asu4, adu4, Wd4, asd4, add4):
    pad_n = NP - N
    Xp = jnp.pad(X1, ((0, pad_n), (0, 0)))
    bp = jnp.pad(batch1, (0, pad_n), constant_values=-1)

    r2 = lambda a: a.reshape(NC * NS * NCH, C)
    usrc, udst = r2(Lu_idx[0]), r2(Lu_idx[1])
    dsrc, ddst = r2(Ld_idx[0]), r2(Ld_idx[1])
    psrc, pdst = r2(L1_idx[1]), r2(L1_idx[0])
    pval = r2(L1_val)

    pw = lambda w: jnp.pad(w, ((0, 0), (0, F4 - OUT)))
    pa = lambda a: jnp.pad(a, (0, F4 - OUT))
    w4 = (pw(Wp4), pw(Wu4), pa(asu4), pa(adu4), pw(Wd4), pa(asd4), pa(add4))

    hp, hu, hd, su, du, sd, dd = _prep1(
        Xp, Wp1, Wu1, asu1, adu1, Wd1, asd1, add1)

    weights = [
        (Wp2, Wu2, asu2, adu2, Wd2, asd2, add2),
        (Wp3, Wu3, asu3, adu3, Wd3, asd3, add3),
        w4,
    ]

    for l in range(4):
        lay = _layer16 if l == 3 else _layer32
        accu, ssu, accd, ssd, accp = lay(
            su, du, sd, dd, hu, hd, hp,
            usrc, udst, dsrc, ddst, psrc, pdst, pval)
        if l < 2:
            hp, hu, hd, su, du, sd, dd = _combine23(
                accu, ssu, accd, ssd, accp, *weights[l])
        elif l == 2:
            hp, hu, hd, su, du, sd, dd = _combine4(
                accu, ssu, accd, ssd, accp, *weights[l])
        else:
            out16 = _pool(accu, ssu, accd, ssd, accp, bp)

    return out16[:, :OUT]
